# baseline (speedup 1.0000x reference)
import functools

import jax
import jax.numpy as jnp
from jax import lax
from jax.experimental import pallas as pl
from jax.experimental.pallas import tpu as pltpu
from jax.experimental.pallas import tpu_sc as plsc

_PAD_IDX = 1
_BLK = 64


def _sc_gather(emb_weight, idx_flat, n_rows, d):
    info = plsc.get_sparse_core_info()
    nw = info.num_cores * info.num_subcores
    rows_per_w = n_rows // nw

    mesh = plsc.VectorSubcoreMesh(core_axis_name="c", subcore_axis_name="s")

    @functools.partial(
        pl.kernel,
        mesh=mesh,
        out_type=jax.ShapeDtypeStruct((n_rows, d), jnp.float32),
        scratch_types=[
            pltpu.VMEM((rows_per_w,), jnp.int32),
            pltpu.VMEM((rows_per_w, d), jnp.float32),
            pltpu.SemaphoreType.DMA,
        ],
    )
    def gather_kernel(emb_hbm, idx_hbm, out_hbm, idx_v, rows_v, sem):
        wid = lax.axis_index("s") * info.num_cores + lax.axis_index("c")
        base = wid * rows_per_w
        pltpu.sync_copy(idx_hbm.at[pl.ds(base, rows_per_w)], idx_v)
        pltpu.async_copy(emb_hbm.at[idx_v], rows_v, sem).wait()
        pltpu.sync_copy(rows_v, out_hbm.at[pl.ds(base, rows_per_w)])

    return gather_kernel(emb_weight, idx_flat)


def _merge_body(prompt_ref, x_ref, o_ref, psem, xsem):
    b, t, d = x_ref.shape
    p = prompt_ref.shape[1]
    pcp = pltpu.make_async_copy(prompt_ref, o_ref.at[:, pl.ds(0, p), :], psem)
    pcp.start()
    xcps = []
    for bi in range(b):
        cp = pltpu.make_async_copy(x_ref.at[bi], o_ref.at[bi, pl.ds(p, t)], xsem)
        cp.start()
        xcps.append(cp)
    pcp.wait()
    for cp in xcps:
        cp.wait()


def kernel(x, encoder_padding_mask, src_prompt, source_prompt_length_padding_mask, emb_weight):
    b, t, d = x.shape
    p = src_prompt.shape[1]

    idx = jnp.where(source_prompt_length_padding_mask, _PAD_IDX, src_prompt)
    idx_flat = idx.astype(jnp.int32).reshape(b * p)

    prompt_rows = _sc_gather(emb_weight, idx_flat, b * p, d)
    prompt_emb = prompt_rows.reshape(b, p, d)

    out = pl.pallas_call(
        _merge_body,
        in_specs=[
            pl.BlockSpec(memory_space=pl.ANY),
            pl.BlockSpec(memory_space=pl.ANY),
        ],
        out_specs=pl.BlockSpec(memory_space=pl.ANY),
        out_shape=jax.ShapeDtypeStruct((b, p + t, d), x.dtype),
        scratch_shapes=[pltpu.SemaphoreType.DMA, pltpu.SemaphoreType.DMA],
    )(prompt_emb, x)

    out_padding_mask = jnp.concatenate(
        [source_prompt_length_padding_mask, encoder_padding_mask], axis=1
    )
    return out, out_padding_mask


# speedup vs baseline: 20.0360x; 20.0360x over previous
import functools

import jax
import jax.numpy as jnp
from jax import lax
from jax.experimental import pallas as pl
from jax.experimental.pallas import tpu as pltpu
from jax.experimental.pallas import tpu_sc as plsc

_PAD_IDX = 1
_BLK = 64


def _sc_gather(emb_weight, idx_flat, n_rows, d):
    info = plsc.get_sparse_core_info()
    nw = info.num_cores * info.num_subcores
    rows_per_w = n_rows // nw

    mesh = plsc.VectorSubcoreMesh(core_axis_name="c", subcore_axis_name="s")

    @functools.partial(
        pl.kernel,
        mesh=mesh,
        out_type=jax.ShapeDtypeStruct((n_rows, d), jnp.float32),
        scratch_types=[
            pltpu.VMEM((rows_per_w,), jnp.int32),
            pltpu.VMEM((rows_per_w, d), jnp.float32),
            pltpu.SemaphoreType.DMA,
        ],
    )
    def gather_kernel(emb_hbm, idx_hbm, out_hbm, idx_v, rows_v, sem):
        wid = lax.axis_index("s") * info.num_cores + lax.axis_index("c")
        base = wid * rows_per_w
        pltpu.sync_copy(idx_hbm.at[pl.ds(base, rows_per_w)], idx_v)
        pltpu.async_copy(emb_hbm.at[idx_v], rows_v, sem).wait()
        pltpu.sync_copy(rows_v, out_hbm.at[pl.ds(base, rows_per_w)])

    return gather_kernel(emb_weight, idx_flat)


_ROWS = 1056


def _merge_body(prompt_hbm, x_hbm, o_ref, sem):
    p = prompt_hbm.shape[1]
    b = pl.program_id(0)
    j = pl.program_id(1)

    @pl.when(j == 0)
    def _():
        cp_p = pltpu.make_async_copy(prompt_hbm.at[b], o_ref.at[0, pl.ds(0, p)], sem)
        cp_x = pltpu.make_async_copy(
            x_hbm.at[b, pl.ds(0, _ROWS - p)], o_ref.at[0, pl.ds(p, _ROWS - p)], sem
        )
        cp_p.start()
        cp_x.start()
        cp_p.wait()
        cp_x.wait()

    @pl.when(j != 0)
    def _():
        cp = pltpu.make_async_copy(
            x_hbm.at[b, pl.ds(j * _ROWS - p, _ROWS)], o_ref.at[0], sem
        )
        cp.start()
        cp.wait()


def kernel(x, encoder_padding_mask, src_prompt, source_prompt_length_padding_mask, emb_weight):
    b, t, d = x.shape
    p = src_prompt.shape[1]

    idx = jnp.where(source_prompt_length_padding_mask, _PAD_IDX, src_prompt)
    idx_flat = idx.astype(jnp.int32).reshape(b * p)

    prompt_rows = _sc_gather(emb_weight, idx_flat, b * p, d)
    prompt_emb = prompt_rows.reshape(b, p, d)

    out = pl.pallas_call(
        _merge_body,
        grid=(b, (p + t) // _ROWS),
        in_specs=[
            pl.BlockSpec(memory_space=pl.ANY),
            pl.BlockSpec(memory_space=pl.ANY),
        ],
        out_specs=pl.BlockSpec((1, _ROWS, d), lambda bi, j: (bi, j, 0)),
        out_shape=jax.ShapeDtypeStruct((b, p + t, d), x.dtype),
        scratch_shapes=[pltpu.SemaphoreType.DMA],
        compiler_params=pltpu.CompilerParams(
            dimension_semantics=("arbitrary", "arbitrary"),
        ),
    )(prompt_emb, x)

    out_padding_mask = jnp.concatenate(
        [source_prompt_length_padding_mask, encoder_padding_mask], axis=1
    )
    return out, out_padding_mask


# speedup vs baseline: 23.0865x; 1.1523x over previous
import functools

import jax
import jax.numpy as jnp
from jax import lax
from jax.experimental import pallas as pl
from jax.experimental.pallas import tpu as pltpu
from jax.experimental.pallas import tpu_sc as plsc

_PAD_IDX = 1
_CHUNK = 32


def _sc_merge(emb_weight, idx_flat, x_flat, b, t, p, d):
    info = plsc.get_sparse_core_info()
    nc = info.num_cores
    nw = nc * info.num_subcores
    ppw = (b * p) // nw
    xpw = (b * t) // nw
    wpb = nw // b
    nch = xpw // _CHUNK

    mesh = plsc.VectorSubcoreMesh(core_axis_name="c", subcore_axis_name="s")

    @functools.partial(
        pl.kernel,
        mesh=mesh,
        out_type=jax.ShapeDtypeStruct((b * (p + t), d), jnp.float32),
        scratch_types=[
            pltpu.VMEM((ppw,), jnp.int32),
            pltpu.VMEM((ppw, d), jnp.float32),
            pltpu.VMEM((_CHUNK, d), jnp.float32),
            pltpu.VMEM((_CHUNK, d), jnp.float32),
            pltpu.SemaphoreType.DMA,
            pltpu.SemaphoreType.DMA,
            pltpu.SemaphoreType.DMA,
        ],
    )
    def merge_kernel(emb_hbm, idx_hbm, x_hbm, out_hbm,
                     idx_v, grows, buf0, buf1, gsem, isem, osem):
        wid = lax.axis_index("s") * nc + lax.axis_index("c")
        bidx = wid // wpb
        wl = wid % wpb

        pltpu.sync_copy(idx_hbm.at[pl.ds(wid * ppw, ppw)], idx_v)
        gcp = pltpu.make_async_copy(emb_hbm.at[idx_v], grows, gsem)
        gcp.start()

        xbase = wid * xpw
        obase = bidx * (p + t) + p + wl * xpw
        bufs = (buf0, buf1)
        in_cps = [None] * nch
        out_cps = [None] * nch
        in_cps[0] = pltpu.make_async_copy(
            x_hbm.at[pl.ds(xbase, _CHUNK)], buf0, isem)
        in_cps[0].start()
        for c in range(nch):
            in_cps[c].wait()
            out_cps[c] = pltpu.make_async_copy(
                bufs[c % 2], out_hbm.at[pl.ds(obase + c * _CHUNK, _CHUNK)], osem)
            out_cps[c].start()
            if c + 1 < nch:
                if c >= 1:
                    out_cps[c - 1].wait()
                in_cps[c + 1] = pltpu.make_async_copy(
                    x_hbm.at[pl.ds(xbase + (c + 1) * _CHUNK, _CHUNK)],
                    bufs[(c + 1) % 2], isem)
                in_cps[c + 1].start()
        if nch >= 2:
            out_cps[nch - 2].wait()
        out_cps[nch - 1].wait()

        gcp.wait()
        gbase = bidx * (p + t) + wl * ppw
        pltpu.sync_copy(grows, out_hbm.at[pl.ds(gbase, ppw)])

    return merge_kernel(emb_weight, idx_flat, x_flat)


def kernel(x, encoder_padding_mask, src_prompt, source_prompt_length_padding_mask, emb_weight):
    b, t, d = x.shape
    p = src_prompt.shape[1]

    idx = jnp.where(source_prompt_length_padding_mask, _PAD_IDX, src_prompt)
    idx_flat = idx.astype(jnp.int32).reshape(b * p)

    out_flat = _sc_merge(emb_weight, idx_flat, x.reshape(b * t, d), b, t, p, d)
    out = out_flat.reshape(b, p + t, d)

    out_padding_mask = jnp.concatenate(
        [source_prompt_length_padding_mask, encoder_padding_mask], axis=1
    )
    return out, out_padding_mask


# speedup vs baseline: 24.7111x; 1.0704x over previous
import functools

import jax
import jax.numpy as jnp
from jax import lax
from jax.experimental import pallas as pl
from jax.experimental.pallas import tpu as pltpu
from jax.experimental.pallas import tpu_sc as plsc

_PAD_IDX = 1


def _sc_gather(emb_weight, idx_flat, n_rows, d):
    info = plsc.get_sparse_core_info()
    nw = info.num_cores * info.num_subcores
    rows_per_w = n_rows // nw

    mesh = plsc.VectorSubcoreMesh(core_axis_name="c", subcore_axis_name="s")

    @functools.partial(
        pl.kernel,
        mesh=mesh,
        out_type=jax.ShapeDtypeStruct((n_rows, d), jnp.float32),
        scratch_types=[
            pltpu.VMEM((rows_per_w,), jnp.int32),
            pltpu.VMEM((rows_per_w, d), jnp.float32),
            pltpu.SemaphoreType.DMA,
        ],
    )
    def gather_kernel(emb_hbm, idx_hbm, out_hbm, idx_v, rows_v, sem):
        wid = lax.axis_index("s") * info.num_cores + lax.axis_index("c")
        base = wid * rows_per_w
        pltpu.sync_copy(idx_hbm.at[pl.ds(base, rows_per_w)], idx_v)
        pltpu.async_copy(emb_hbm.at[idx_v], rows_v, sem).wait()
        pltpu.sync_copy(rows_v, out_hbm.at[pl.ds(base, rows_per_w)])

    return gather_kernel(emb_weight, idx_flat)


def _merge_body(p_ref, x_ref, o_ref):
    p = p_ref.shape[1]
    t = x_ref.shape[1]
    o_ref[0, 0:p] = p_ref[0]
    o_ref[0, p : p + t] = x_ref[0]


def kernel(x, encoder_padding_mask, src_prompt, source_prompt_length_padding_mask, emb_weight):
    b, t, d = x.shape
    p = src_prompt.shape[1]

    idx = jnp.where(source_prompt_length_padding_mask, _PAD_IDX, src_prompt)
    idx_flat = idx.astype(jnp.int32).reshape(b * p)

    prompt_rows = _sc_gather(emb_weight, idx_flat, b * p, d)
    prompt_emb = prompt_rows.reshape(b, p, d)

    out = pl.pallas_call(
        _merge_body,
        grid=(b,),
        in_specs=[
            pl.BlockSpec((1, p, d), lambda bi: (bi, 0, 0)),
            pl.BlockSpec((1, t, d), lambda bi: (bi, 0, 0)),
        ],
        out_specs=pl.BlockSpec((1, p + t, d), lambda bi: (bi, 0, 0)),
        out_shape=jax.ShapeDtypeStruct((b, p + t, d), x.dtype),
        compiler_params=pltpu.CompilerParams(
            dimension_semantics=("arbitrary",),
        ),
    )(prompt_emb, x)

    out_padding_mask = jnp.concatenate(
        [source_prompt_length_padding_mask, encoder_padding_mask], axis=1
    )
    return out, out_padding_mask


# speedup vs baseline: 25.1134x; 1.0163x over previous
import functools

import jax
import jax.numpy as jnp
from jax import lax
from jax.experimental import pallas as pl
from jax.experimental.pallas import tpu as pltpu
from jax.experimental.pallas import tpu_sc as plsc

_PAD_IDX = 1


def _sc_gather(emb_weight, idx_flat, n_rows, d):
    info = plsc.get_sparse_core_info()
    nw = 1 * info.num_subcores
    rows_per_w = n_rows // nw

    mesh = plsc.VectorSubcoreMesh(core_axis_name="c", subcore_axis_name="s", num_cores=1)

    @functools.partial(
        pl.kernel,
        mesh=mesh,
        out_type=jax.ShapeDtypeStruct((n_rows, d), jnp.float32),
        scratch_types=[
            pltpu.VMEM((rows_per_w,), jnp.int32),
            pltpu.VMEM((rows_per_w, d), jnp.float32),
            pltpu.SemaphoreType.DMA,
        ],
    )
    def gather_kernel(emb_hbm, idx_hbm, out_hbm, idx_v, rows_v, sem):
        wid = lax.axis_index("s")
        base = wid * rows_per_w
        pltpu.sync_copy(idx_hbm.at[pl.ds(base, rows_per_w)], idx_v)
        pltpu.async_copy(emb_hbm.at[idx_v], rows_v, sem).wait()
        pltpu.sync_copy(rows_v, out_hbm.at[pl.ds(base, rows_per_w)])

    return gather_kernel(emb_weight, idx_flat)


def _merge_body(p_ref, x_ref, o_ref):
    p = p_ref.shape[1]
    t = x_ref.shape[1]
    o_ref[0, 0:p] = p_ref[0]
    o_ref[0, p : p + t] = x_ref[0]


def kernel(x, encoder_padding_mask, src_prompt, source_prompt_length_padding_mask, emb_weight):
    b, t, d = x.shape
    p = src_prompt.shape[1]

    idx = jnp.where(source_prompt_length_padding_mask, _PAD_IDX, src_prompt)
    idx_flat = idx.astype(jnp.int32).reshape(b * p)

    prompt_rows = _sc_gather(emb_weight, idx_flat, b * p, d)
    prompt_emb = prompt_rows.reshape(b, p, d)

    out = pl.pallas_call(
        _merge_body,
        grid=(b,),
        in_specs=[
            pl.BlockSpec((1, p, d), lambda bi: (bi, 0, 0)),
            pl.BlockSpec((1, t, d), lambda bi: (bi, 0, 0)),
        ],
        out_specs=pl.BlockSpec((1, p + t, d), lambda bi: (bi, 0, 0)),
        out_shape=jax.ShapeDtypeStruct((b, p + t, d), x.dtype),
        compiler_params=pltpu.CompilerParams(
            dimension_semantics=("arbitrary",),
        ),
    )(prompt_emb, x)

    out_padding_mask = jnp.concatenate(
        [source_prompt_length_padding_mask, encoder_padding_mask], axis=1
    )
    return out, out_padding_mask
